# baseline (speedup 1.0000x reference)
import functools

import jax
import jax.numpy as jnp
import numpy as np
from jax import lax
from jax.experimental import pallas as pl
from jax.experimental.pallas import tpu as pltpu
from jax.experimental.pallas import tpu_sc as plsc

_BATCH = 16384
_N_FIELDS = 26
_EMBED_DIM = 16
_VOCAB_PER_FIELD = 40000
_TOTAL_ROWS = _BATCH * _N_FIELDS

_NUM_CORES = 2
_NUM_SUBCORES = 16
_NUM_WORKERS = _NUM_CORES * _NUM_SUBCORES
_ROWS_PER_WORKER = _TOTAL_ROWS // _NUM_WORKERS

_LANES = 16
_PERIOD = 208
_PERIOD_VECS = _PERIOD // _LANES
_GROUPS_PER_WORKER = _ROWS_PER_WORKER // _PERIOD

_CHUNK = 3328
_NUM_CHUNKS = _ROWS_PER_WORKER // _CHUNK

_OFFSET_PATTERN = (np.arange(_PERIOD) % _N_FIELDS).astype(np.int32) * _VOCAB_PER_FIELD

@functools.lru_cache(maxsize=1)
def _build_gather_kernel():
    mesh = plsc.VectorSubcoreMesh(core_axis_name="c", subcore_axis_name="s")

    @functools.partial(
        pl.kernel,
        mesh=mesh,
        out_type=jax.ShapeDtypeStruct((_TOTAL_ROWS, _EMBED_DIM), jnp.float32),
        scratch_types=[
            pltpu.VMEM((_ROWS_PER_WORKER,), jnp.int32),
            pltpu.VMEM((_PERIOD,), jnp.int32),
            pltpu.VMEM((_CHUNK, _EMBED_DIM), jnp.float32),
            pltpu.SemaphoreType.DMA,
        ],
        compiler_params=pltpu.CompilerParams(use_tc_tiling_on_sc=False),
    )
    def gather_kernel(x_hbm, pat_hbm, table_hbm, out_hbm, idx_v, pat_v, buf, sem):
        wid = lax.axis_index("s") * _NUM_CORES + lax.axis_index("c")
        base = wid * _ROWS_PER_WORKER

        pltpu.sync_copy(x_hbm.at[pl.ds(base, _ROWS_PER_WORKER)], idx_v)
        pltpu.sync_copy(pat_hbm, pat_v)

        def add_group(g, carry):
            for j in range(_PERIOD_VECS):
                s = pl.ds(g * _PERIOD + j * _LANES, _LANES)
                idx_v[s] = idx_v[s] + pat_v[pl.ds(j * _LANES, _LANES)]
            return carry

        lax.fori_loop(0, _GROUPS_PER_WORKER, add_group, 0)

        for c in range(_NUM_CHUNKS):
            idx_view = idx_v.at[pl.ds(c * _CHUNK, _CHUNK)]
            pltpu.async_copy(table_hbm.at[idx_view], buf, sem).wait()
            pltpu.sync_copy(buf, out_hbm.at[pl.ds(base + c * _CHUNK, _CHUNK)])

    return gather_kernel


def kernel(x, table):
    x_flat = x.reshape(_TOTAL_ROWS)
    out = _build_gather_kernel()(x_flat, jnp.asarray(_OFFSET_PATTERN), table)
    return out.reshape(_BATCH, _N_FIELDS, _EMBED_DIM)


# speedup vs baseline: 1.1264x; 1.1264x over previous
import functools

import jax
import jax.numpy as jnp
from jax import lax
from jax.experimental import pallas as pl
from jax.experimental.pallas import tpu as pltpu
from jax.experimental.pallas import tpu_sc as plsc

_B = 16384
_F = 26
_E = 16
_VPF = 40000
_V = _F * _VPF
_NC = 2
_NS = 16
_NW = _NC * _NS
_L = 16

_NTILES = _V // 128
_TILES_PER_W = -(-_NTILES // _NW)
_SLAB_ROWS = _V // 8

_BPW = _B // _NW
_IVECS = _BPW // _L


@functools.lru_cache(maxsize=1)
def _build_kernels():
    mesh = plsc.VectorSubcoreMesh(core_axis_name="c", subcore_axis_name="s")
    params = pltpu.CompilerParams(needs_layout_passes=False)

    @functools.partial(
        pl.kernel,
        mesh=mesh,
        out_type=jax.ShapeDtypeStruct((_SLAB_ROWS, 128), jnp.float32),
        scratch_types=[
            pltpu.VMEM((_E, 128), jnp.float32),
            pltpu.VMEM((_E, 128), jnp.float32),
        ],
        compiler_params=params,
    )
    def untile_kernel(tabt_hbm, slab_hbm, tilebuf, rowbuf):
        w = lax.axis_index("s") * _NC + lax.axis_index("c")
        iota = lax.iota(jnp.int32, _L)

        def tile_body(k, carry):
            j = w + k * _NW

            @pl.when(j < _NTILES)
            def _():
                pltpu.sync_copy(tabt_hbm.at[:, pl.ds(j * 128, 128)], tilebuf)

                def col_body(g, carry2):
                    for m in range(8):
                        vec = plsc.load_gather(
                            tilebuf, [iota, jnp.full((_L,), g * 8 + m, jnp.int32)]
                        )
                        rowbuf[g, pl.ds(m * _L, _L)] = vec
                    return carry2

                lax.fori_loop(0, _E, col_body, 0)
                pltpu.sync_copy(rowbuf, slab_hbm.at[pl.ds(j * _E, _E), :])

            return carry

        lax.fori_loop(0, _TILES_PER_W, tile_body, 0)

    @functools.partial(
        pl.kernel,
        mesh=mesh,
        out_type=jax.ShapeDtypeStruct((_F, _E, _B), jnp.float32),
        scratch_types=[
            pltpu.VMEM((_BPW,), jnp.int32),
            pltpu.VMEM((_BPW,), jnp.int32),
            pltpu.VMEM((_BPW, 128), jnp.float32),
            pltpu.VMEM((_E, _BPW), jnp.float32),
            pltpu.SemaphoreType.DMA,
        ],
        compiler_params=params,
    )
    def gather_kernel(xt_hbm, slab_hbm, out_hbm, gidx, cbase, gbuf, obuf, sem):
        w = lax.axis_index("s") * _NC + lax.axis_index("c")
        b0 = w * _BPW
        iota = lax.iota(jnp.int32, _L)

        for f in range(_F):
            off = f * _VPF
            pltpu.sync_copy(xt_hbm.at[f, pl.ds(b0, _BPW)], gidx)

            def idx_body(j, carry):
                sl = pl.ds(j * _L, _L)
                u = gidx[sl] + off
                cbase[sl] = (u & 7) * _L
                gidx[sl] = u >> 3
                return carry

            lax.fori_loop(0, _IVECS, idx_body, 0, unroll=4)

            pltpu.async_copy(slab_hbm.at[gidx], gbuf, sem).wait()

            def trans_body(j, carry):
                rows = j * _L + iota
                cvec = cbase[pl.ds(j * _L, _L)]
                for e in range(_E):
                    vec = plsc.load_gather(gbuf, [rows, cvec + e])
                    obuf[e, pl.ds(j * _L, _L)] = vec
                return carry

            lax.fori_loop(0, _IVECS, trans_body, 0)

            pltpu.sync_copy(obuf, out_hbm.at[f, :, pl.ds(b0, _BPW)])

    return untile_kernel, gather_kernel


def kernel(x, table):
    untile_kernel, gather_kernel = _build_kernels()
    x_t = x.T
    tab_t = table.T
    slab = untile_kernel(tab_t)
    out_t = gather_kernel(x_t, slab)
    return jnp.transpose(out_t, (2, 0, 1))


# speedup vs baseline: 3.0716x; 2.7270x over previous
import functools

import jax
import jax.numpy as jnp
from jax import lax
from jax.experimental import pallas as pl
from jax.experimental.pallas import tpu as pltpu
from jax.experimental.pallas import tpu_sc as plsc

_B = 16384
_F = 26
_E = 16
_VPF = 40000
_V = _F * _VPF
_NC = 2
_NS = 16
_NW = _NC * _NS
_L = 16

_NTILES = _V // 128
_K1_ITERS = -(-_NTILES // _NW)
_SLAB_ROWS = _V // 8

_BPW = _B // _NW
_BPH = _BPW // 2
_HIVECS = _BPH // _L


@functools.lru_cache(maxsize=1)
def _build_kernels():
    mesh = plsc.VectorSubcoreMesh(core_axis_name="c", subcore_axis_name="s")
    params = pltpu.CompilerParams(needs_layout_passes=False)

    @functools.partial(
        pl.kernel,
        mesh=mesh,
        out_type=jax.ShapeDtypeStruct((_SLAB_ROWS, 128), jnp.float32),
        scratch_types=[
            pltpu.VMEM((_E, 128), jnp.float32),
            pltpu.VMEM((_E, 128), jnp.float32),
            pltpu.VMEM((_E, 128), jnp.float32),
            pltpu.VMEM((_E, 128), jnp.float32),
            pltpu.SemaphoreType.DMA,
            pltpu.SemaphoreType.DMA,
            pltpu.SemaphoreType.DMA,
            pltpu.SemaphoreType.DMA,
        ],
        compiler_params=params,
    )
    def untile_kernel(tabt_hbm, slab_hbm, tb0, tb1, rb0, rb1, is0, is1, os0, os1):
        w = lax.axis_index("s") * _NC + lax.axis_index("c")
        iota = lax.iota(jnp.int32, _L)
        tbufs, rbufs, isems, osems = (tb0, tb1), (rb0, rb1), (is0, is1), (os0, os1)

        rows_c = [(jnp.full((_L,), 16 * c, jnp.int32) + iota) >> 3 for c in range(8)]
        colb_c = [((jnp.full((_L,), 16 * c, jnp.int32) + iota) & 7) * _L
                  for c in range(8)]

        def start_in(k, par):
            j = w + k * _NW
            @pl.when(j < _NTILES)
            def _():
                pltpu.async_copy(
                    tabt_hbm.at[:, pl.ds(j * 128, 128)], tbufs[par], isems[par]
                )

        start_in(0, 0)
        start_in(1, 1)

        def tile_pair(kk, carry):
            for par in range(2):
                k = kk * 2 + par
                j = w + k * _NW

                @pl.when(j < _NTILES)
                def _():
                    @pl.when(k >= 2)
                    def _():
                        pltpu.make_async_copy(
                            rbufs[par], slab_hbm.at[pl.ds(0, _E), :], osems[par]
                        ).wait()
                    pltpu.make_async_copy(
                        tabt_hbm.at[:, pl.ds(0, 128)], tbufs[par], isems[par]
                    ).wait()
                    for c in range(8):
                        for e in range(_E):
                            vec = tbufs[par][e, pl.ds(c * _L, _L)]
                            plsc.store_scatter(
                                rbufs[par], [rows_c[c], colb_c[c] + e], vec
                            )
                    pltpu.async_copy(
                        rbufs[par], slab_hbm.at[pl.ds(j * _E, _E), :], osems[par]
                    )
                    start_in(k + 2, par)
            return carry

        lax.fori_loop(0, _K1_ITERS // 2, tile_pair, 0)

        for par in range(2):
            pltpu.make_async_copy(
                rbufs[par], slab_hbm.at[pl.ds(0, _E), :], osems[par]
            ).wait()

    @functools.partial(
        pl.kernel,
        mesh=mesh,
        out_type=jax.ShapeDtypeStruct((_F, _E, _B), jnp.float32),
        scratch_types=[
            pltpu.VMEM((_F, _BPW), jnp.int32),
            pltpu.VMEM((_BPH,), jnp.int32),
            pltpu.VMEM((_BPH,), jnp.int32),
            pltpu.VMEM((_BPH,), jnp.int32),
            pltpu.VMEM((_BPH,), jnp.int32),
            pltpu.VMEM((_BPH, 128), jnp.float32),
            pltpu.VMEM((_BPH, 128), jnp.float32),
            pltpu.VMEM((_E, _BPH), jnp.float32),
            pltpu.VMEM((_E, _BPH), jnp.float32),
            pltpu.SemaphoreType.DMA,
            pltpu.SemaphoreType.DMA,
            pltpu.SemaphoreType.DMA,
            pltpu.SemaphoreType.DMA,
        ],
        compiler_params=params,
    )
    def gather_kernel(xt_hbm, slab_hbm, out_hbm, xall, gi0, gi1, cb0, cb1,
                      gb0, gb1, ob0, ob1, gs0, gs1, vs0, vs1):
        w = lax.axis_index("s") * _NC + lax.axis_index("c")
        b0 = w * _BPW
        iota = lax.iota(jnp.int32, _L)
        gidxs, cbases = (gi0, gi1), (cb0, cb1)
        gbufs, obufs = (gb0, gb1), (ob0, ob1)
        gsems, vsems = (gs0, gs1), (vs0, vs1)

        pltpu.sync_copy(xt_hbm.at[:, pl.ds(b0, _BPW)], xall)

        def start_gather(f, h):
            @pl.when(f < _F)
            def _():
                off = f * _VPF

                def idx_body(j, carry):
                    sl = pl.ds(h * _BPH + j * _L, _L)
                    dl = pl.ds(j * _L, _L)
                    u = xall[f, sl] + off
                    cbases[h][dl] = (u & 7) * _L
                    gidxs[h][dl] = u >> 3
                    return carry

                lax.fori_loop(0, _HIVECS, idx_body, 0, unroll=4)
                pltpu.async_copy(slab_hbm.at[gidxs[h]], gbufs[h], gsems[h])

        start_gather(0, 0)
        start_gather(0, 1)

        def field_body(f, carry):
            for h in range(2):
                pltpu.make_async_copy(
                    slab_hbm.at[gidxs[h]], gbufs[h], gsems[h]
                ).wait()

                @pl.when(f >= 1)
                def _():
                    pltpu.make_async_copy(
                        obufs[h], out_hbm.at[0, :, pl.ds(0, _BPH)], vsems[h]
                    ).wait()

                def trans_body(j, carry2):
                    rows = j * _L + iota
                    cvec = cbases[h][pl.ds(j * _L, _L)]
                    for e in range(_E):
                        vec = plsc.load_gather(gbufs[h], [rows, cvec + e])
                        obufs[h][e, pl.ds(j * _L, _L)] = vec
                    return carry2

                lax.fori_loop(0, _HIVECS, trans_body, 0)

                pltpu.async_copy(
                    obufs[h],
                    out_hbm.at[f, :, pl.ds(b0 + h * _BPH, _BPH)],
                    vsems[h],
                )
                start_gather(f + 1, h)
            return carry

        lax.fori_loop(0, _F, field_body, 0)

        for h in range(2):
            pltpu.make_async_copy(
                obufs[h], out_hbm.at[0, :, pl.ds(0, _BPH)], vsems[h]
            ).wait()

    return untile_kernel, gather_kernel


def kernel(x, table):
    untile_kernel, gather_kernel = _build_kernels()
    x_t = x.T
    tab_t = table.T
    slab = untile_kernel(tab_t)
    out_t = gather_kernel(x_t, slab)
    return jnp.transpose(out_t, (2, 0, 1))


# speedup vs baseline: 3.2191x; 1.0480x over previous
import functools

import jax
import jax.numpy as jnp
from jax import lax
from jax.experimental import pallas as pl
from jax.experimental.pallas import tpu as pltpu
from jax.experimental.pallas import tpu_sc as plsc

_B = 16384
_F = 26
_E = 16
_VPF = 40000
_V = _F * _VPF
_NC = 2
_NS = 16
_NW = _NC * _NS
_L = 16

_NTILES = _V // 128
_GT = 4
_NGROUPS = _NTILES // _GT
_K1_ITERS = -(-_NGROUPS // _NW)
_SLAB_ROWS = _V // 8

_BPW = _B // _NW
_NQ = 4
_BPQ = _BPW // _NQ
_QIVECS = _BPQ // _L


@functools.lru_cache(maxsize=1)
def _build_kernels():
    mesh = plsc.VectorSubcoreMesh(core_axis_name="c", subcore_axis_name="s")
    params = pltpu.CompilerParams(needs_layout_passes=False)

    @functools.partial(
        pl.kernel,
        mesh=mesh,
        out_type=jax.ShapeDtypeStruct((_SLAB_ROWS, 128), jnp.float32),
        scratch_types=[
            pltpu.VMEM((_E, _GT * 128), jnp.float32),
            pltpu.VMEM((_E, _GT * 128), jnp.float32),
            pltpu.VMEM((_GT * _E, 128), jnp.float32),
            pltpu.VMEM((_GT * _E, 128), jnp.float32),
            pltpu.SemaphoreType.DMA,
            pltpu.SemaphoreType.DMA,
            pltpu.SemaphoreType.DMA,
            pltpu.SemaphoreType.DMA,
        ],
        compiler_params=params,
    )
    def untile_kernel(tabt_hbm, slab_hbm, tb0, tb1, rb0, rb1, is0, is1, os0, os1):
        w = lax.axis_index("s") * _NC + lax.axis_index("c")
        iota = lax.iota(jnp.int32, _L)
        tbufs, rbufs, isems, osems = (tb0, tb1), (rb0, rb1), (is0, is1), (os0, os1)

        rows_tc = [[((jnp.full((_L,), 16 * c, jnp.int32) + iota) >> 3) + 16 * t
                    for c in range(8)] for t in range(_GT)]
        colb_c = [((jnp.full((_L,), 16 * c, jnp.int32) + iota) & 7) * _L
                  for c in range(8)]

        def start_in(k, par):
            g = w + k * _NW
            @pl.when(g < _NGROUPS)
            def _():
                pltpu.async_copy(
                    tabt_hbm.at[:, pl.ds(g * _GT * 128, _GT * 128)],
                    tbufs[par], isems[par],
                )

        start_in(0, 0)
        start_in(1, 1)

        def group_pair(kk, carry):
            for par in range(2):
                k = kk * 2 + par
                g = w + k * _NW

                @pl.when(g < _NGROUPS)
                def _():
                    @pl.when(k >= 2)
                    def _():
                        pltpu.make_async_copy(
                            rbufs[par], slab_hbm.at[pl.ds(0, _GT * _E), :],
                            osems[par],
                        ).wait()
                    pltpu.make_async_copy(
                        tabt_hbm.at[:, pl.ds(0, _GT * 128)], tbufs[par],
                        isems[par],
                    ).wait()
                    for t in range(_GT):
                        for c in range(8):
                            for e in range(_E):
                                vec = tbufs[par][e, pl.ds(t * 128 + c * _L, _L)]
                                plsc.store_scatter(
                                    rbufs[par], [rows_tc[t][c], colb_c[c] + e],
                                    vec,
                                )
                    pltpu.async_copy(
                        rbufs[par],
                        slab_hbm.at[pl.ds(g * _GT * _E, _GT * _E), :],
                        osems[par],
                    )
                    start_in(k + 2, par)
            return carry

        lax.fori_loop(0, _K1_ITERS // 2, group_pair, 0)

        for par in range(2):
            pltpu.make_async_copy(
                rbufs[par], slab_hbm.at[pl.ds(0, _GT * _E), :], osems[par]
            ).wait()

        @pl.when(w == 0)
        def _():
            jt = _NGROUPS * _GT
            pltpu.sync_copy(tabt_hbm.at[:, pl.ds(jt * 128, 128)],
                            tb0.at[:, pl.ds(0, 128)])
            for c in range(8):
                for e in range(_E):
                    vec = tb0[e, pl.ds(c * _L, _L)]
                    plsc.store_scatter(
                        rb0.at[pl.ds(0, _E), :], [rows_tc[0][c], colb_c[c] + e],
                        vec,
                    )
            pltpu.sync_copy(rb0.at[pl.ds(0, _E), :],
                            slab_hbm.at[pl.ds(jt * _E, _E), :])

    @functools.partial(
        pl.kernel,
        mesh=mesh,
        out_type=jax.ShapeDtypeStruct((_F, _E, _B), jnp.float32),
        scratch_types=[
            pltpu.VMEM((_F, _BPW), jnp.int32),
            [pltpu.VMEM((_BPQ,), jnp.int32) for _ in range(_NQ)],
            [pltpu.VMEM((_BPQ,), jnp.int32) for _ in range(_NQ)],
            [pltpu.VMEM((_BPQ, 128), jnp.float32) for _ in range(_NQ)],
            [pltpu.VMEM((_E, _BPQ), jnp.float32) for _ in range(_NQ)],
            [pltpu.SemaphoreType.DMA for _ in range(_NQ)],
            [pltpu.SemaphoreType.DMA for _ in range(_NQ)],
        ],
        compiler_params=params,
    )
    def gather_kernel(xt_hbm, slab_hbm, out_hbm, xall, gidxs, cbases,
                      gbufs, obufs, gsems, vsems):
        w = lax.axis_index("s") * _NC + lax.axis_index("c")
        b0 = w * _BPW
        iota = lax.iota(jnp.int32, _L)

        pltpu.sync_copy(xt_hbm.at[:, pl.ds(b0, _BPW)], xall)

        def start_gather(f, q):
            @pl.when(f < _F)
            def _():
                off = f * _VPF

                def idx_body(j, carry):
                    sl = pl.ds(q * _BPQ + j * _L, _L)
                    dl = pl.ds(j * _L, _L)
                    u = xall[f, sl] + off
                    cbases[q][dl] = (u & 7) * _L
                    gidxs[q][dl] = u >> 3
                    return carry

                lax.fori_loop(0, _QIVECS, idx_body, 0, unroll=4)
                pltpu.async_copy(slab_hbm.at[gidxs[q]], gbufs[q], gsems[q])

        for q in range(_NQ):
            start_gather(0, q)

        def field_body(f, carry):
            for q in range(_NQ):
                pltpu.make_async_copy(
                    slab_hbm.at[gidxs[q]], gbufs[q], gsems[q]
                ).wait()

                @pl.when(f >= 1)
                def _():
                    pltpu.make_async_copy(
                        obufs[q], out_hbm.at[0, :, pl.ds(0, _BPQ)], vsems[q]
                    ).wait()

                def trans_body(j, carry2):
                    rows = j * _L + iota
                    cvec = cbases[q][pl.ds(j * _L, _L)]
                    for e in range(_E):
                        vec = plsc.load_gather(gbufs[q], [rows, cvec + e])
                        obufs[q][e, pl.ds(j * _L, _L)] = vec
                    return carry2

                lax.fori_loop(0, _QIVECS, trans_body, 0)

                pltpu.async_copy(
                    obufs[q],
                    out_hbm.at[f, :, pl.ds(b0 + q * _BPQ, _BPQ)],
                    vsems[q],
                )
                start_gather(f + 1, q)
            return carry

        lax.fori_loop(0, _F, field_body, 0)

        for q in range(_NQ):
            pltpu.make_async_copy(
                obufs[q], out_hbm.at[0, :, pl.ds(0, _BPQ)], vsems[q]
            ).wait()

    return untile_kernel, gather_kernel


def kernel(x, table):
    untile_kernel, gather_kernel = _build_kernels()
    x_t = x.T
    tab_t = table.T
    slab = untile_kernel(tab_t)
    out_t = gather_kernel(x_t, slab)
    return jnp.transpose(out_t, (2, 0, 1))
